# baseline (speedup 1.0000x reference)
import jax
import jax.numpy as jnp
from jax.experimental import pallas as pl
from jax.experimental.pallas import tpu as pltpu


def _deg_body(adj_ref, deg_ref, adjq_ref):
    a = adj_ref[:]
    deg_ref[:] = jnp.sum(a, axis=1, keepdims=True)
    adjq_ref[:] = a.astype(jnp.float8_e4m3fn)


def _main_body(adjq_ref, xs_ref, x0_ref, d_ref, dinv_ref, inv_s_ref,
               w0_ref, w1_ref, w2_ref, bc_ref, wo_ref, b2_ref,
               o_ref, y1_scr, s_scr):
    p = pl.program_id(0)
    i = pl.program_id(1)
    r = adjq_ref.shape[0]

    @pl.when(p == 0)
    def _phase_y1():
        x1 = (-d_ref[:] * inv_s_ref[:]) * jax.lax.dot_general(
            adjq_ref[:], xs_ref[:], (((1,), (0,)), ((), ())),
            preferred_element_type=jnp.float32)
        y1_scr[pl.ds(i * r, r), :] = (d_ref[:] * x1).astype(jnp.bfloat16)

    @pl.when(p == 1)
    def _phase_support():
        x0 = x0_ref[:]
        x2 = (-2.0 * d_ref[:] * jax.lax.dot_general(
            adjq_ref[:], y1_scr[:], (((1,), (0,)), ((), ())),
            preferred_element_type=jnp.float32)
              - x0)
        y1_blk = y1_scr[pl.ds(i * r, r), :].astype(jnp.float32)
        h = (jnp.dot(x0, w0_ref[:], preferred_element_type=jnp.float32)
             + dinv_ref[:] * jnp.dot(y1_blk, w1_ref[:],
                                     preferred_element_type=jnp.float32)
             + jnp.dot(x2, w2_ref[:], preferred_element_type=jnp.float32)
             + bc_ref[:])
        h = jnp.maximum(h, 0.0)
        s_scr[pl.ds(i * r, r), :] = jnp.dot(
            h, wo_ref[:], preferred_element_type=jnp.float32
        ).astype(jnp.bfloat16)

    @pl.when(p == 2)
    def _phase_pool():
        part = jax.lax.dot_general(
            adjq_ref[:], s_scr[:], (((1,), (0,)), ((), ())),
            preferred_element_type=jnp.float32)
        m = jnp.max(part, axis=0, keepdims=True) + b2_ref[:]

        @pl.when(i == 0)
        def _init():
            o_ref[:] = m

        @pl.when(i != 0)
        def _acc():
            o_ref[:] = jnp.maximum(o_ref[:], m)


def kernel(x, adj, W_cheb, b_cheb, W2, b2):
    N, F = x.shape
    H = W_cheb.shape[2]
    C = W2.shape[1]
    R1 = next((r for r in (400, 200, 80, 40, 16, 8) if N % r == 0), N)
    R2 = next((r for r in (80, 16) if N % r == 0), N)

    deg, adjq = pl.pallas_call(
        _deg_body,
        grid=(N // R1,),
        in_specs=[pl.BlockSpec((R1, N), lambda i: (i, 0))],
        out_specs=[pl.BlockSpec((R1, 1), lambda i: (i, 0)),
                   pl.BlockSpec((R1, N), lambda i: (i, 0))],
        out_shape=[jax.ShapeDtypeStruct((N, 1), jnp.float32),
                   jax.ShapeDtypeStruct((N, N), jnp.float8_e4m3fn)],
    )(adj)

    d = jnp.where(deg > 0, jax.lax.rsqrt(jnp.maximum(deg, 1e-12)), 0.0)
    dinv = jnp.where(deg > 0, jnp.sqrt(jnp.maximum(deg, 1e-12)), 0.0)
    xs_f32 = x * d
    amax = jnp.maximum(jnp.max(jnp.abs(xs_f32)), 1e-30)
    scale = jnp.exp2(jnp.floor(jnp.log2(64.0 / amax)))
    inv_s = (1.0 / scale).reshape(1, 1)
    xs = (xs_f32 * scale).astype(jnp.float8_e4m3fn)

    pooled = pl.pallas_call(
        _main_body,
        grid=(3, N // R2),
        in_specs=[
            pl.BlockSpec((R2, N), lambda p, i: (i, 0)),
            pl.BlockSpec((N, F), lambda p, i: (0, 0)),
            pl.BlockSpec((R2, F), lambda p, i: (jnp.where(p == 1, i, 0), 0)),
            pl.BlockSpec((R2, 1), lambda p, i: (i, 0)),
            pl.BlockSpec((R2, 1), lambda p, i: (i, 0)),
            pl.BlockSpec((1, 1), lambda p, i: (0, 0)),
            pl.BlockSpec((F, H), lambda p, i: (0, 0)),
            pl.BlockSpec((F, H), lambda p, i: (0, 0)),
            pl.BlockSpec((F, H), lambda p, i: (0, 0)),
            pl.BlockSpec((1, H), lambda p, i: (0, 0)),
            pl.BlockSpec((H, C), lambda p, i: (0, 0)),
            pl.BlockSpec((1, C), lambda p, i: (0, 0)),
        ],
        out_specs=pl.BlockSpec((1, C), lambda p, i: (0, 0)),
        out_shape=jax.ShapeDtypeStruct((1, C), jnp.float32),
        scratch_shapes=[pltpu.VMEM((N, F), jnp.bfloat16),
                        pltpu.VMEM((N, C), jnp.bfloat16)],
    )(adjq, xs, x, d, dinv, inv_s, W_cheb[0], W_cheb[1], W_cheb[2],
      b_cheb.reshape(1, H), W2, b2.reshape(1, C))

    return pooled[None, :, :]


# speedup vs baseline: 1.5625x; 1.5625x over previous
import jax
import jax.numpy as jnp
from jax.experimental import pallas as pl


def _deg_body(adj_ref, deg_ref, adjq_ref):
    a = adj_ref[:]
    deg_ref[:] = jnp.sum(a, axis=1, keepdims=True)
    adjq_ref[:] = a.astype(jnp.float8_e4m3fn)


def _f8dot(a, b):
    return jax.lax.dot_general(a, b, (((1,), (0,)), ((), ())),
                               preferred_element_type=jnp.float32)


def _y1_body(adjq_ref, xs_ref, d_ref, inv_s_ref, y1_ref):
    x1 = (-d_ref[:] * inv_s_ref[:]) * _f8dot(adjq_ref[:], xs_ref[:])
    y1_ref[:] = d_ref[:] * x1


def _supp_body(adjq_ref, y1q_ref, x0_ref, d_ref, dinv_ref, inv_sy_ref,
               w0_ref, w1_ref, w2_ref, bc_ref, wo_ref, s_ref):
    i = pl.program_id(0)
    r = x0_ref.shape[0]
    x0 = x0_ref[:]
    inv_sy = inv_sy_ref[:]
    x2 = (-2.0 * d_ref[:] * inv_sy) * _f8dot(adjq_ref[:], y1q_ref[:]) - x0
    y1_blk = (y1q_ref[pl.ds(i * r, r), :].astype(jnp.float32) * inv_sy)
    h = (jnp.dot(x0, w0_ref[:], preferred_element_type=jnp.float32)
         + dinv_ref[:] * jnp.dot(y1_blk, w1_ref[:],
                                 preferred_element_type=jnp.float32)
         + jnp.dot(x2, w2_ref[:], preferred_element_type=jnp.float32)
         + bc_ref[:])
    h = jnp.maximum(h, 0.0)
    s_ref[:] = jnp.dot(h, wo_ref[:],
                       preferred_element_type=jnp.float32).astype(jnp.bfloat16)


def _pool_body(adjq_ref, s_ref, o_ref):
    part = _f8dot(adjq_ref[:], s_ref[:])
    o_ref[:] = jnp.max(part, axis=0, keepdims=True)[None, :, :]


def _pow2_scale(a, target=64.0):
    amax = jnp.maximum(jnp.max(jnp.abs(a)), 1e-30)
    return jnp.exp2(jnp.floor(jnp.log2(target / amax)))


def kernel(x, adj, W_cheb, b_cheb, W2, b2):
    N, F = x.shape
    H = W_cheb.shape[2]
    C = W2.shape[1]
    R1 = next((r for r in (400, 200, 80, 40, 16, 8) if N % r == 0), N)
    R2 = next((r for r in (1000, 400, 80, 16, 8) if N % r == 0), N)
    R3 = next((r for r in (400, 80, 16, 8) if N % r == 0), N)

    deg, adjq = pl.pallas_call(
        _deg_body,
        grid=(N // R1,),
        in_specs=[pl.BlockSpec((R1, N), lambda i: (i, 0))],
        out_specs=[pl.BlockSpec((R1, 1), lambda i: (i, 0)),
                   pl.BlockSpec((R1, N), lambda i: (i, 0))],
        out_shape=[jax.ShapeDtypeStruct((N, 1), jnp.float32),
                   jax.ShapeDtypeStruct((N, N), jnp.float8_e4m3fn)],
    )(adj)

    d = jnp.where(deg > 0, jax.lax.rsqrt(jnp.maximum(deg, 1e-12)), 0.0)
    dinv = jnp.where(deg > 0, jnp.sqrt(jnp.maximum(deg, 1e-12)), 0.0)
    xs_f32 = x * d
    s_xs = _pow2_scale(xs_f32)
    xs = (xs_f32 * s_xs).astype(jnp.float8_e4m3fn)
    inv_sxs = (1.0 / s_xs).reshape(1, 1)

    y1 = pl.pallas_call(
        _y1_body,
        grid=(N // R2,),
        in_specs=[
            pl.BlockSpec((R2, N), lambda i: (i, 0)),
            pl.BlockSpec((N, F), lambda i: (0, 0)),
            pl.BlockSpec((R2, 1), lambda i: (i, 0)),
            pl.BlockSpec((1, 1), lambda i: (0, 0)),
        ],
        out_specs=pl.BlockSpec((R2, F), lambda i: (i, 0)),
        out_shape=jax.ShapeDtypeStruct((N, F), jnp.float32),
    )(adjq, xs, d, inv_sxs)

    s_y1 = _pow2_scale(y1)
    y1q = (y1 * s_y1).astype(jnp.float8_e4m3fn)
    inv_sy1 = (1.0 / s_y1).reshape(1, 1)

    support = pl.pallas_call(
        _supp_body,
        grid=(N // R3,),
        in_specs=[
            pl.BlockSpec((R3, N), lambda i: (i, 0)),
            pl.BlockSpec((N, F), lambda i: (0, 0)),
            pl.BlockSpec((R3, F), lambda i: (i, 0)),
            pl.BlockSpec((R3, 1), lambda i: (i, 0)),
            pl.BlockSpec((R3, 1), lambda i: (i, 0)),
            pl.BlockSpec((1, 1), lambda i: (0, 0)),
            pl.BlockSpec((F, H), lambda i: (0, 0)),
            pl.BlockSpec((F, H), lambda i: (0, 0)),
            pl.BlockSpec((F, H), lambda i: (0, 0)),
            pl.BlockSpec((1, H), lambda i: (0, 0)),
            pl.BlockSpec((H, C), lambda i: (0, 0)),
        ],
        out_specs=pl.BlockSpec((R3, C), lambda i: (i, 0)),
        out_shape=jax.ShapeDtypeStruct((N, C), jnp.bfloat16),
    )(adjq, y1q, x, d, dinv, inv_sy1, W_cheb[0], W_cheb[1], W_cheb[2],
      b_cheb.reshape(1, H), W2)

    partials = pl.pallas_call(
        _pool_body,
        grid=(N // R3,),
        in_specs=[
            pl.BlockSpec((R3, N), lambda i: (i, 0)),
            pl.BlockSpec((N, C), lambda i: (0, 0)),
        ],
        out_specs=pl.BlockSpec((1, 1, C), lambda i: (i, 0, 0)),
        out_shape=jax.ShapeDtypeStruct((N // R3, 1, C), jnp.float32),
    )(adjq, support)

    pooled = jnp.max(partials, axis=0) + b2[None, :]
    return pooled[None, :, :]


# speedup vs baseline: 1.5868x; 1.0155x over previous
import jax
import jax.numpy as jnp
from jax.experimental import pallas as pl


def _deg_body(adj_ref, deg_ref, adjq_ref):
    a = adj_ref[:]
    deg_ref[:] = jnp.sum(a, axis=1, keepdims=True)
    adjq_ref[:] = a.astype(jnp.float8_e4m3fn)


def _f8dot(a, b):
    return jax.lax.dot_general(a, b, (((1,), (0,)), ((), ())),
                               preferred_element_type=jnp.float32)


def _y1_body(adjq_ref, xs_ref, d_ref, inv_s_ref, y1_ref):
    x1 = (-d_ref[:] * inv_s_ref[:]) * _f8dot(adjq_ref[:], xs_ref[:])
    y1_ref[:] = d_ref[:] * x1


def _supp_body(adjq_ref, y1q_ref, x0_ref, d_ref, dinv_ref, inv_sy_ref,
               w0_ref, w1_ref, w2_ref, bc_ref, wo_ref, s_ref):
    i = pl.program_id(0)
    r = x0_ref.shape[0]
    x0 = x0_ref[:]
    inv_sy = inv_sy_ref[:]
    x2 = (-2.0 * d_ref[:] * inv_sy) * _f8dot(adjq_ref[:], y1q_ref[:]) - x0
    y1_blk = (y1q_ref[pl.ds(i * r, r), :].astype(jnp.float32) * inv_sy)
    h = (jnp.dot(x0, w0_ref[:], preferred_element_type=jnp.float32)
         + dinv_ref[:] * jnp.dot(y1_blk, w1_ref[:],
                                 preferred_element_type=jnp.float32)
         + jnp.dot(x2, w2_ref[:], preferred_element_type=jnp.float32)
         + bc_ref[:])
    h = jnp.maximum(h, 0.0)
    s_ref[:] = jnp.dot(h, wo_ref[:],
                       preferred_element_type=jnp.float32).astype(jnp.bfloat16)


def _pool_body(adjq_ref, s_ref, o_ref):
    part = _f8dot(adjq_ref[:], s_ref[:])
    o_ref[:] = jnp.max(part, axis=0, keepdims=True)[None, :, :]


def _pow2_scale(a, target=64.0):
    amax = jnp.maximum(jnp.max(jnp.abs(a)), 1e-30)
    return jnp.exp2(jnp.floor(jnp.log2(target / amax)))


def kernel(x, adj, W_cheb, b_cheb, W2, b2):
    N, F = x.shape
    H = W_cheb.shape[2]
    C = W2.shape[1]
    R1 = next((r for r in (400, 200, 80, 40, 16, 8) if N % r == 0), N)
    R2 = next((r for r in (1000, 400, 80, 16, 8) if N % r == 0), N)
    R3 = next((r for r in (1000, 400, 80, 16, 8) if N % r == 0), N)
    R4 = next((r for r in (400, 80, 16, 8) if N % r == 0), N)

    deg, adjq = pl.pallas_call(
        _deg_body,
        grid=(N // R1,),
        in_specs=[pl.BlockSpec((R1, N), lambda i: (i, 0))],
        out_specs=[pl.BlockSpec((R1, 1), lambda i: (i, 0)),
                   pl.BlockSpec((R1, N), lambda i: (i, 0))],
        out_shape=[jax.ShapeDtypeStruct((N, 1), jnp.float32),
                   jax.ShapeDtypeStruct((N, N), jnp.float8_e4m3fn)],
    )(adj)

    d = jnp.where(deg > 0, jax.lax.rsqrt(jnp.maximum(deg, 1e-12)), 0.0)
    dinv = jnp.where(deg > 0, jnp.sqrt(jnp.maximum(deg, 1e-12)), 0.0)
    xs_f32 = x * d
    s_xs = _pow2_scale(xs_f32)
    xs = (xs_f32 * s_xs).astype(jnp.float8_e4m3fn)
    inv_sxs = (1.0 / s_xs).reshape(1, 1)

    y1 = pl.pallas_call(
        _y1_body,
        grid=(N // R2,),
        in_specs=[
            pl.BlockSpec((R2, N), lambda i: (i, 0)),
            pl.BlockSpec((N, F), lambda i: (0, 0)),
            pl.BlockSpec((R2, 1), lambda i: (i, 0)),
            pl.BlockSpec((1, 1), lambda i: (0, 0)),
        ],
        out_specs=pl.BlockSpec((R2, F), lambda i: (i, 0)),
        out_shape=jax.ShapeDtypeStruct((N, F), jnp.float32),
    )(adjq, xs, d, inv_sxs)

    s_y1 = _pow2_scale(y1)
    y1q = (y1 * s_y1).astype(jnp.float8_e4m3fn)
    inv_sy1 = (1.0 / s_y1).reshape(1, 1)

    support = pl.pallas_call(
        _supp_body,
        grid=(N // R3,),
        in_specs=[
            pl.BlockSpec((R3, N), lambda i: (i, 0)),
            pl.BlockSpec((N, F), lambda i: (0, 0)),
            pl.BlockSpec((R3, F), lambda i: (i, 0)),
            pl.BlockSpec((R3, 1), lambda i: (i, 0)),
            pl.BlockSpec((R3, 1), lambda i: (i, 0)),
            pl.BlockSpec((1, 1), lambda i: (0, 0)),
            pl.BlockSpec((F, H), lambda i: (0, 0)),
            pl.BlockSpec((F, H), lambda i: (0, 0)),
            pl.BlockSpec((F, H), lambda i: (0, 0)),
            pl.BlockSpec((1, H), lambda i: (0, 0)),
            pl.BlockSpec((H, C), lambda i: (0, 0)),
        ],
        out_specs=pl.BlockSpec((R3, C), lambda i: (i, 0)),
        out_shape=jax.ShapeDtypeStruct((N, C), jnp.bfloat16),
    )(adjq, y1q, x, d, dinv, inv_sy1, W_cheb[0], W_cheb[1], W_cheb[2],
      b_cheb.reshape(1, H), W2)

    partials = pl.pallas_call(
        _pool_body,
        grid=(N // R4,),
        in_specs=[
            pl.BlockSpec((R4, N), lambda i: (i, 0)),
            pl.BlockSpec((N, C), lambda i: (0, 0)),
        ],
        out_specs=pl.BlockSpec((1, 1, C), lambda i: (i, 0, 0)),
        out_shape=jax.ShapeDtypeStruct((N // R4, 1, C), jnp.float32),
    )(adjq, support)

    pooled = jnp.max(partials, axis=0) + b2[None, :]
    return pooled[None, :, :]
